# Initial kernel scaffold; baseline (speedup 1.0000x reference)
import jax
import jax.numpy as jnp
from jax.experimental import pallas as pl


def kernel(x, edge_index, W1, b1, W2, b2, W3, b3, Wm1, bm1, Wm2, bm2, Wm3, bm3, Wm4, bm4, Wf, bf):
    raise NotImplementedError("write your pallas kernel here")


# speedup vs baseline: 6.9833x; 6.9833x over previous
import functools

import jax
import jax.numpy as jnp
from jax import lax
from jax.experimental import pallas as pl
from jax.experimental.pallas import tpu as pltpu
from jax.experimental.pallas import tpu_sc as plsc

N = 10000
E = 160000
D = 256
QW = 64
DOUT = 128

NP = 10240
ROWS_PER_TILE = NP // 16
K = 128
CH = 79
EP = 16 * CH * K

_mesh = plsc.VectorSubcoreMesh(core_axis_name="c", subcore_axis_name="s")
_SC_PARAMS = pltpu.CompilerParams(use_tc_tiling_on_sc=False)


@functools.partial(
    pl.kernel,
    out_type=jax.ShapeDtypeStruct((NP, 16), jnp.float32),
    mesh=_mesh,
    scratch_types=[
        pltpu.VMEM((CH, K), jnp.int32),
        pltpu.VMEM((K, 16), jnp.float32),
        pltpu.VMEM((ROWS_PER_TILE, 16), jnp.float32),
        pltpu.VMEM_SHARED((NP, 16), jnp.float32),
    ],
    compiler_params=_SC_PARAMS,
)
def _deg_kernel(dst_hbm, out_hbm, dst_v, ones_v, stage_v, accd):
    c = lax.axis_index("c")
    s = lax.axis_index("s")
    r0 = s * ROWS_PER_TILE

    @pl.when(c == 0)
    def _():
        @pl.loop(0, K)
        def _(i):
            ones_v[i, :] = jnp.ones((16,), jnp.float32)

        @pl.loop(0, ROWS_PER_TILE)
        def _(i):
            stage_v[i, :] = jnp.zeros((16,), jnp.float32)

        pltpu.sync_copy(dst_hbm.at[s], dst_v)
        pltpu.sync_copy(stage_v, accd.at[pl.ds(r0, ROWS_PER_TILE)])
        plsc.subcore_barrier()

        @pl.loop(0, CH)
        def _(j):
            pltpu.sync_copy(ones_v, accd.at[dst_v.at[j]], add=True)

        plsc.subcore_barrier()
        pltpu.sync_copy(accd.at[pl.ds(r0, ROWS_PER_TILE)], stage_v)
        pltpu.sync_copy(stage_v, out_hbm.at[pl.ds(r0, ROWS_PER_TILE)])


_QTY = jax.ShapeDtypeStruct((NP, QW), jnp.float32)


@functools.partial(
    pl.kernel,
    out_type=[_QTY, _QTY, _QTY, _QTY],
    mesh=_mesh,
    scratch_types=[
        pltpu.VMEM((CH, K), jnp.int32),
        pltpu.VMEM((CH, K), jnp.int32),
        pltpu.VMEM((K, QW), jnp.float32),
        pltpu.VMEM((ROWS_PER_TILE, QW), jnp.float32),
        pltpu.VMEM_SHARED((NP, QW), jnp.float32),
    ],
    compiler_params=_SC_PARAMS,
)
def _conv_kernel(h0_hbm, h1_hbm, h2_hbm, h3_hbm, src_hbm, dsti_hbm,
                 o0_hbm, o1_hbm, o2_hbm, o3_hbm,
                 src_v, dst_v, rows_v, stage_v, acc):
    c = lax.axis_index("c")
    s = lax.axis_index("s")
    r0 = s * ROWS_PER_TILE
    rows = pl.ds(r0, ROWS_PER_TILE)

    def init(h_hbm):
        pltpu.sync_copy(h_hbm.at[rows], stage_v)
        pltpu.sync_copy(stage_v, acc.at[rows])

    def scatter(h_hbm):
        @pl.loop(0, CH)
        def _(j):
            pltpu.sync_copy(h_hbm.at[src_v.at[j]], rows_v)
            pltpu.sync_copy(rows_v, acc.at[dst_v.at[j]], add=True)

    def writeback(o_hbm):
        pltpu.sync_copy(acc.at[rows], stage_v)
        pltpu.sync_copy(stage_v, o_hbm.at[rows])

    def run(ha, hb, oa, ob):
        pltpu.sync_copy(src_hbm.at[s], src_v)
        pltpu.sync_copy(dsti_hbm.at[s], dst_v)
        init(ha)
        plsc.subcore_barrier()
        scatter(ha)
        plsc.subcore_barrier()
        writeback(oa)
        init(hb)
        plsc.subcore_barrier()
        scatter(hb)
        plsc.subcore_barrier()
        writeback(ob)

    @pl.when(c == 0)
    def _():
        run(h0_hbm, h1_hbm, o0_hbm, o1_hbm)

    @pl.when(c == 1)
    def _():
        run(h2_hbm, h3_hbm, o2_hbm, o3_hbm)


_RB = 1024
_GRID = NP // _RB
_QOUT = [jax.ShapeDtypeStruct((NP, QW), jnp.float32) for _ in range(4)]
_QOUT_SPECS = [pl.BlockSpec((_RB, QW), lambda i: (i, 0)) for _ in range(4)]


def _split_q(h, refs):
    for q, ref in enumerate(refs):
        ref[...] = h[:, q * QW:(q + 1) * QW]


def _t0_body(x_ref, w_ref, deg_ref, *o_refs):
    dinv = lax.rsqrt(deg_ref[:, 0:1] + 1.0)
    h = jnp.dot(x_ref[...], w_ref[...], preferred_element_type=jnp.float32)
    _split_q(h * dinv, o_refs)


def _t0(x_pad, w, deg):
    return pl.pallas_call(
        _t0_body,
        grid=(_GRID,),
        in_specs=[
            pl.BlockSpec((_RB, D), lambda i: (i, 0)),
            pl.BlockSpec((D, D), lambda i: (0, 0)),
            pl.BlockSpec((_RB, 16), lambda i: (i, 0)),
        ],
        out_specs=_QOUT_SPECS,
        out_shape=_QOUT,
    )(x_pad, w, deg)


def _tmid_body(a0, a1, a2, a3, deg_ref, b_ref, w_ref, *o_refs):
    dinv = lax.rsqrt(deg_ref[:, 0:1] + 1.0)
    a = jnp.concatenate([a0[...], a1[...], a2[...], a3[...]], axis=1)
    g = jnp.maximum(a * dinv + b_ref[...], 0.0)
    h = jnp.dot(g, w_ref[...], preferred_element_type=jnp.float32)
    _split_q(h * dinv, o_refs)


def _tmid(aq, deg, b, w):
    return pl.pallas_call(
        _tmid_body,
        grid=(_GRID,),
        in_specs=[
            *_QOUT_SPECS,
            pl.BlockSpec((_RB, 16), lambda i: (i, 0)),
            pl.BlockSpec((1, D), lambda i: (0, 0)),
            pl.BlockSpec((D, D), lambda i: (0, 0)),
        ],
        out_specs=_QOUT_SPECS,
        out_shape=_QOUT,
    )(*aq, deg, b, w)


def _tail_body(a0, a1, a2, a3, deg_ref, b3_ref,
               wm1_ref, bm1_ref, wm2_ref, bm2_ref,
               wm3_ref, bm3_ref, wm4_ref, bm4_ref,
               wf_ref, bf_ref, out_ref):
    dinv = lax.rsqrt(deg_ref[:, 0:1] + 1.0)
    a = jnp.concatenate([a0[...], a1[...], a2[...], a3[...]], axis=1)
    g = jnp.maximum(a * dinv + b3_ref[...], 0.0)
    for w_ref, b_ref in ((wm1_ref, bm1_ref), (wm2_ref, bm2_ref),
                         (wm3_ref, bm3_ref), (wm4_ref, bm4_ref)):
        g = jnp.dot(g, w_ref[...], preferred_element_type=jnp.float32)
        g = jnp.maximum(g + b_ref[...], 0.0)
    out = jnp.dot(g, wf_ref[...], preferred_element_type=jnp.float32)
    out_ref[...] = out + bf_ref[...]


def _tail(aq, deg, b3, wm1, bm1, wm2, bm2, wm3, bm3, wm4, bm4, wf, bf):
    full = lambda r, cdim: pl.BlockSpec((r, cdim), lambda i: (0, 0))
    return pl.pallas_call(
        _tail_body,
        grid=(_GRID,),
        in_specs=[
            *_QOUT_SPECS,
            pl.BlockSpec((_RB, 16), lambda i: (i, 0)),
            full(1, D),
            full(D, D), full(1, D), full(D, D), full(1, D),
            full(D, D), full(1, D), full(D, D), full(1, D),
            full(D, DOUT), full(1, DOUT),
        ],
        out_specs=pl.BlockSpec((_RB, DOUT), lambda i: (i, 0)),
        out_shape=jax.ShapeDtypeStruct((NP, DOUT), jnp.float32),
    )(*aq, deg, b3, wm1, bm1, wm2, bm2, wm3, bm3, wm4, bm4, wf, bf)


def kernel(x, edge_index, W1, b1, W2, b2, W3, b3,
           Wm1, bm1, Wm2, bm2, Wm3, bm3, Wm4, bm4, Wf, bf):
    x_pad = jnp.pad(x, ((0, NP - N), (0, 0)))
    pad = jnp.full((EP - E,), N, dtype=jnp.int32)
    srcp = jnp.concatenate([edge_index[0], pad]).reshape(16, CH, K)
    dstp = jnp.concatenate([edge_index[1], pad]).reshape(16, CH, K)

    deg = _deg_kernel(dstp)

    hq = _t0(x_pad, W1, deg)
    aq = _conv_kernel(*hq, srcp, dstp)
    hq = _tmid(aq, deg, b1.reshape(1, D), W2)
    aq = _conv_kernel(*hq, srcp, dstp)
    hq = _tmid(aq, deg, b2.reshape(1, D), W3)
    aq = _conv_kernel(*hq, srcp, dstp)
    out = _tail(aq, deg, b3.reshape(1, D),
                Wm1, bm1.reshape(1, D), Wm2, bm2.reshape(1, D),
                Wm3, bm3.reshape(1, D), Wm4, bm4.reshape(1, D),
                Wf, bf.reshape(1, DOUT))
    return out[:N]


# speedup vs baseline: 7.1844x; 1.0288x over previous
import functools

import jax
import jax.numpy as jnp
from jax import lax
from jax.experimental import pallas as pl
from jax.experimental.pallas import tpu as pltpu
from jax.experimental.pallas import tpu_sc as plsc

N = 10000
E = 160000
D = 256
QW = 64
DOUT = 128

NP = 10240
ROWS_PER_TILE = NP // 16
K = 128
CH = 80
EP = 16 * CH * K

_mesh = plsc.VectorSubcoreMesh(core_axis_name="c", subcore_axis_name="s")
_SC_PARAMS = pltpu.CompilerParams(use_tc_tiling_on_sc=False)


@functools.partial(
    pl.kernel,
    out_type=jax.ShapeDtypeStruct((NP, 16), jnp.float32),
    mesh=_mesh,
    scratch_types=[
        pltpu.VMEM((CH, K), jnp.int32),
        pltpu.VMEM((K, 16), jnp.float32),
        pltpu.VMEM((ROWS_PER_TILE, 16), jnp.float32),
        pltpu.VMEM_SHARED((NP, 16), jnp.float32),
    ],
    compiler_params=_SC_PARAMS,
)
def _deg_kernel(dst_hbm, out_hbm, dst_v, ones_v, stage_v, accd):
    c = lax.axis_index("c")
    s = lax.axis_index("s")
    r0 = s * ROWS_PER_TILE

    @pl.when(c == 0)
    def _():
        @pl.loop(0, K)
        def _(i):
            ones_v[i, :] = jnp.ones((16,), jnp.float32)

        @pl.loop(0, ROWS_PER_TILE)
        def _(i):
            stage_v[i, :] = jnp.zeros((16,), jnp.float32)

        pltpu.sync_copy(dst_hbm.at[s], dst_v)
        pltpu.sync_copy(stage_v, accd.at[pl.ds(r0, ROWS_PER_TILE)])
        plsc.subcore_barrier()

        @pl.loop(0, CH)
        def _(j):
            pltpu.sync_copy(ones_v, accd.at[dst_v.at[j]], add=True)

        plsc.subcore_barrier()
        pltpu.sync_copy(accd.at[pl.ds(r0, ROWS_PER_TILE)], stage_v)
        pltpu.sync_copy(stage_v, out_hbm.at[pl.ds(r0, ROWS_PER_TILE)])


_QTY = jax.ShapeDtypeStruct((NP, QW), jnp.float32)


@functools.partial(
    pl.kernel,
    out_type=[_QTY, _QTY, _QTY, _QTY],
    mesh=_mesh,
    scratch_types=[
        pltpu.VMEM((CH, K), jnp.int32),
        pltpu.VMEM((CH, K), jnp.int32),
        pltpu.VMEM((K, QW), jnp.float32),
        pltpu.VMEM((K, QW), jnp.float32),
        pltpu.VMEM((ROWS_PER_TILE, QW), jnp.float32),
        pltpu.VMEM_SHARED((NP, QW), jnp.float32),
        pltpu.SemaphoreType.DMA,
        pltpu.SemaphoreType.DMA,
    ],
    compiler_params=_SC_PARAMS,
)
def _conv_kernel(h0_hbm, h1_hbm, h2_hbm, h3_hbm, src_hbm, dsti_hbm,
                 o0_hbm, o1_hbm, o2_hbm, o3_hbm,
                 src_v, dst_v, rows_a, rows_b, stage_v, acc,
                 gsem_a, gsem_b):
    c = lax.axis_index("c")
    s = lax.axis_index("s")
    r0 = s * ROWS_PER_TILE
    rows = pl.ds(r0, ROWS_PER_TILE)

    def init(h_hbm):
        pltpu.sync_copy(h_hbm.at[rows], stage_v)
        pltpu.sync_copy(stage_v, acc.at[rows])

    def scatter(h_hbm):
        pltpu.async_copy(h_hbm.at[src_v.at[0]], rows_a, gsem_a)
        pltpu.async_copy(h_hbm.at[src_v.at[1]], rows_b, gsem_b)

        @pl.loop(0, CH // 2)
        def _(t):
            j = 2 * t
            pltpu.make_async_copy(h_hbm.at[src_v.at[j]], rows_a, gsem_a).wait()
            pltpu.sync_copy(rows_a, acc.at[dst_v.at[j]], add=True)

            @pl.when(t < CH // 2 - 1)
            def _():
                pltpu.async_copy(h_hbm.at[src_v.at[j + 2]], rows_a, gsem_a)

            pltpu.make_async_copy(h_hbm.at[src_v.at[j + 1]], rows_b,
                                  gsem_b).wait()
            pltpu.sync_copy(rows_b, acc.at[dst_v.at[j + 1]], add=True)

            @pl.when(t < CH // 2 - 1)
            def _():
                pltpu.async_copy(h_hbm.at[src_v.at[j + 3]], rows_b, gsem_b)

    def writeback(o_hbm):
        pltpu.sync_copy(acc.at[rows], stage_v)
        pltpu.sync_copy(stage_v, o_hbm.at[rows])

    def run(ha, hb, oa, ob):
        pltpu.sync_copy(src_hbm.at[s], src_v)
        pltpu.sync_copy(dsti_hbm.at[s], dst_v)
        init(ha)
        plsc.subcore_barrier()
        scatter(ha)
        plsc.subcore_barrier()
        writeback(oa)
        init(hb)
        plsc.subcore_barrier()
        scatter(hb)
        plsc.subcore_barrier()
        writeback(ob)

    @pl.when(c == 0)
    def _():
        run(h0_hbm, h1_hbm, o0_hbm, o1_hbm)

    @pl.when(c == 1)
    def _():
        run(h2_hbm, h3_hbm, o2_hbm, o3_hbm)


_RB = 1024
_GRID = NP // _RB
_QOUT = [jax.ShapeDtypeStruct((NP, QW), jnp.float32) for _ in range(4)]
_QOUT_SPECS = [pl.BlockSpec((_RB, QW), lambda i: (i, 0)) for _ in range(4)]


def _split_q(h, refs):
    for q, ref in enumerate(refs):
        ref[...] = h[:, q * QW:(q + 1) * QW]


def _t0_body(x_ref, w_ref, deg_ref, *o_refs):
    dinv = lax.rsqrt(deg_ref[:, 0:1] + 1.0)
    h = jnp.dot(x_ref[...], w_ref[...], preferred_element_type=jnp.float32)
    _split_q(h * dinv, o_refs)


def _t0(x_pad, w, deg):
    return pl.pallas_call(
        _t0_body,
        grid=(_GRID,),
        in_specs=[
            pl.BlockSpec((_RB, D), lambda i: (i, 0)),
            pl.BlockSpec((D, D), lambda i: (0, 0)),
            pl.BlockSpec((_RB, 16), lambda i: (i, 0)),
        ],
        out_specs=_QOUT_SPECS,
        out_shape=_QOUT,
    )(x_pad, w, deg)


def _tmid_body(a0, a1, a2, a3, deg_ref, b_ref, w_ref, *o_refs):
    dinv = lax.rsqrt(deg_ref[:, 0:1] + 1.0)
    a = jnp.concatenate([a0[...], a1[...], a2[...], a3[...]], axis=1)
    g = jnp.maximum(a * dinv + b_ref[...], 0.0)
    h = jnp.dot(g, w_ref[...], preferred_element_type=jnp.float32)
    _split_q(h * dinv, o_refs)


def _tmid(aq, deg, b, w):
    return pl.pallas_call(
        _tmid_body,
        grid=(_GRID,),
        in_specs=[
            *_QOUT_SPECS,
            pl.BlockSpec((_RB, 16), lambda i: (i, 0)),
            pl.BlockSpec((1, D), lambda i: (0, 0)),
            pl.BlockSpec((D, D), lambda i: (0, 0)),
        ],
        out_specs=_QOUT_SPECS,
        out_shape=_QOUT,
    )(*aq, deg, b, w)


def _tail_body(a0, a1, a2, a3, deg_ref, b3_ref,
               wm1_ref, bm1_ref, wm2_ref, bm2_ref,
               wm3_ref, bm3_ref, wm4_ref, bm4_ref,
               wf_ref, bf_ref, out_ref):
    dinv = lax.rsqrt(deg_ref[:, 0:1] + 1.0)
    a = jnp.concatenate([a0[...], a1[...], a2[...], a3[...]], axis=1)
    g = jnp.maximum(a * dinv + b3_ref[...], 0.0)
    for w_ref, b_ref in ((wm1_ref, bm1_ref), (wm2_ref, bm2_ref),
                         (wm3_ref, bm3_ref), (wm4_ref, bm4_ref)):
        g = jnp.dot(g, w_ref[...], preferred_element_type=jnp.float32)
        g = jnp.maximum(g + b_ref[...], 0.0)
    out = jnp.dot(g, wf_ref[...], preferred_element_type=jnp.float32)
    out_ref[...] = out + bf_ref[...]


def _tail(aq, deg, b3, wm1, bm1, wm2, bm2, wm3, bm3, wm4, bm4, wf, bf):
    full = lambda r, cdim: pl.BlockSpec((r, cdim), lambda i: (0, 0))
    return pl.pallas_call(
        _tail_body,
        grid=(_GRID,),
        in_specs=[
            *_QOUT_SPECS,
            pl.BlockSpec((_RB, 16), lambda i: (i, 0)),
            full(1, D),
            full(D, D), full(1, D), full(D, D), full(1, D),
            full(D, D), full(1, D), full(D, D), full(1, D),
            full(D, DOUT), full(1, DOUT),
        ],
        out_specs=pl.BlockSpec((_RB, DOUT), lambda i: (i, 0)),
        out_shape=jax.ShapeDtypeStruct((NP, DOUT), jnp.float32),
    )(*aq, deg, b3, wm1, bm1, wm2, bm2, wm3, bm3, wm4, bm4, wf, bf)


def kernel(x, edge_index, W1, b1, W2, b2, W3, b3,
           Wm1, bm1, Wm2, bm2, Wm3, bm3, Wm4, bm4, Wf, bf):
    x_pad = jnp.pad(x, ((0, NP - N), (0, 0)))
    pad = jnp.full((EP - E,), N, dtype=jnp.int32)
    srcp = jnp.concatenate([edge_index[0], pad]).reshape(16, CH, K)
    dstp = jnp.concatenate([edge_index[1], pad]).reshape(16, CH, K)

    deg = _deg_kernel(dstp)

    hq = _t0(x_pad, W1, deg)
    aq = _conv_kernel(*hq, srcp, dstp)
    hq = _tmid(aq, deg, b1.reshape(1, D), W2)
    aq = _conv_kernel(*hq, srcp, dstp)
    hq = _tmid(aq, deg, b2.reshape(1, D), W3)
    aq = _conv_kernel(*hq, srcp, dstp)
    out = _tail(aq, deg, b3.reshape(1, D),
                Wm1, bm1.reshape(1, D), Wm2, bm2.reshape(1, D),
                Wm3, bm3.reshape(1, D), Wm4, bm4.reshape(1, D),
                Wf, bf.reshape(1, DOUT))
    return out[:N]


# speedup vs baseline: 7.2382x; 1.0075x over previous
import functools

import jax
import jax.numpy as jnp
from jax import lax
from jax.experimental import pallas as pl
from jax.experimental.pallas import tpu as pltpu
from jax.experimental.pallas import tpu_sc as plsc

N = 10000
E = 160000
D = 256
QW = 64
DOUT = 128

NP = 10240
ROWS_PER_TILE = NP // 16
K = 128
CH = 80
EP = 16 * CH * K

_mesh = plsc.VectorSubcoreMesh(core_axis_name="c", subcore_axis_name="s")
_SC_PARAMS = pltpu.CompilerParams(use_tc_tiling_on_sc=False)


@functools.partial(
    pl.kernel,
    out_type=jax.ShapeDtypeStruct((NP, 16), jnp.float32),
    mesh=_mesh,
    scratch_types=[
        pltpu.VMEM((CH, K), jnp.int32),
        pltpu.VMEM((K, 16), jnp.float32),
        pltpu.VMEM((ROWS_PER_TILE, 16), jnp.float32),
        pltpu.VMEM_SHARED((NP, 16), jnp.float32),
    ],
    compiler_params=_SC_PARAMS,
)
def _deg_kernel(dst_hbm, out_hbm, dst_v, ones_v, stage_v, accd):
    c = lax.axis_index("c")
    s = lax.axis_index("s")
    r0 = s * ROWS_PER_TILE

    @pl.when(c == 0)
    def _():
        @pl.loop(0, K)
        def _(i):
            ones_v[i, :] = jnp.ones((16,), jnp.float32)

        @pl.loop(0, ROWS_PER_TILE)
        def _(i):
            stage_v[i, :] = jnp.zeros((16,), jnp.float32)

        pltpu.sync_copy(dst_hbm.at[s], dst_v)
        pltpu.sync_copy(stage_v, accd.at[pl.ds(r0, ROWS_PER_TILE)])
        plsc.subcore_barrier()

        @pl.loop(0, CH)
        def _(j):
            pltpu.sync_copy(ones_v, accd.at[dst_v.at[j]], add=True)

        plsc.subcore_barrier()
        pltpu.sync_copy(accd.at[pl.ds(r0, ROWS_PER_TILE)], stage_v)
        pltpu.sync_copy(stage_v, out_hbm.at[pl.ds(r0, ROWS_PER_TILE)])


_QTY = jax.ShapeDtypeStruct((NP, QW), jnp.float32)


@functools.partial(
    pl.kernel,
    out_type=[_QTY, _QTY, _QTY, _QTY],
    mesh=_mesh,
    scratch_types=[
        pltpu.VMEM((CH, K), jnp.int32),
        pltpu.VMEM((CH, K), jnp.int32),
        pltpu.VMEM((K, QW), jnp.float32),
        pltpu.VMEM((K, QW), jnp.float32),
        pltpu.VMEM((K, QW), jnp.float32),
        pltpu.VMEM((K, QW), jnp.float32),
        pltpu.VMEM_SHARED((NP, QW), jnp.float32),
        pltpu.SemaphoreType.DMA,
        pltpu.SemaphoreType.DMA,
        pltpu.SemaphoreType.DMA,
        pltpu.SemaphoreType.DMA,
        pltpu.SemaphoreType.DMA,
        pltpu.SemaphoreType.DMA,
        pltpu.SemaphoreType.DMA,
        pltpu.SemaphoreType.DMA,
    ],
    compiler_params=_SC_PARAMS,
)
def _conv_kernel(h0_hbm, h1_hbm, h2_hbm, h3_hbm, src_hbm, dsti_hbm,
                 o0_hbm, o1_hbm, o2_hbm, o3_hbm,
                 src_v, dst_v, rows_a, rows_b, rows_c, rows_d, acc,
                 gsem_a, gsem_b, gsem_c, gsem_d,
                 ssem_a, ssem_b, ssem_c, ssem_d):
    c = lax.axis_index("c")
    s = lax.axis_index("s")
    r0 = s * ROWS_PER_TILE
    rows = pl.ds(r0, ROWS_PER_TILE)

    def init(h_hbm):
        pltpu.sync_copy(h_hbm.at[rows], acc.at[rows])

    def scatter(h_hbm):
        ring = ((rows_a, gsem_a, ssem_a), (rows_b, gsem_b, ssem_b),
                (rows_c, gsem_c, ssem_c), (rows_d, gsem_d, ssem_d))
        R = len(ring)

        for r, (buf, gsem, _) in enumerate(ring):
            pltpu.async_copy(h_hbm.at[src_v.at[r]], buf, gsem)

        @pl.loop(0, CH // R)
        def _(t):
            j0 = R * t
            for r, (buf, gsem, ssem) in enumerate(ring):
                pltpu.make_async_copy(h_hbm.at[src_v.at[j0 + r]], buf,
                                      gsem).wait()
                pltpu.async_copy(buf, acc.at[dst_v.at[j0 + r]], ssem,
                                 add=True)
            for r, (buf, gsem, ssem) in enumerate(ring):
                pltpu.make_async_copy(buf, acc.at[dst_v.at[j0 + r]],
                                      ssem).wait()

                @pl.when(t < CH // R - 1)
                def _():
                    pltpu.async_copy(h_hbm.at[src_v.at[j0 + R + r]], buf,
                                     gsem)

    def writeback(o_hbm):
        pltpu.sync_copy(acc.at[rows], o_hbm.at[rows])

    def run(ha, hb, oa, ob):
        pltpu.sync_copy(src_hbm.at[s], src_v)
        pltpu.sync_copy(dsti_hbm.at[s], dst_v)
        init(ha)
        plsc.subcore_barrier()
        scatter(ha)
        plsc.subcore_barrier()
        writeback(oa)
        init(hb)
        plsc.subcore_barrier()
        scatter(hb)
        plsc.subcore_barrier()
        writeback(ob)

    @pl.when(c == 0)
    def _():
        run(h0_hbm, h1_hbm, o0_hbm, o1_hbm)

    @pl.when(c == 1)
    def _():
        run(h2_hbm, h3_hbm, o2_hbm, o3_hbm)


_RB = 1024
_GRID = NP // _RB
_QOUT = [jax.ShapeDtypeStruct((NP, QW), jnp.float32) for _ in range(4)]
_QOUT_SPECS = [pl.BlockSpec((_RB, QW), lambda i: (i, 0)) for _ in range(4)]


def _split_q(h, refs):
    for q, ref in enumerate(refs):
        ref[...] = h[:, q * QW:(q + 1) * QW]


def _t0_body(x_ref, w_ref, deg_ref, *o_refs):
    dinv = lax.rsqrt(deg_ref[:, 0:1] + 1.0)
    h = jnp.dot(x_ref[...], w_ref[...], preferred_element_type=jnp.float32)
    _split_q(h * dinv, o_refs)


def _t0(x_pad, w, deg):
    return pl.pallas_call(
        _t0_body,
        grid=(_GRID,),
        in_specs=[
            pl.BlockSpec((_RB, D), lambda i: (i, 0)),
            pl.BlockSpec((D, D), lambda i: (0, 0)),
            pl.BlockSpec((_RB, 16), lambda i: (i, 0)),
        ],
        out_specs=_QOUT_SPECS,
        out_shape=_QOUT,
    )(x_pad, w, deg)


def _tmid_body(a0, a1, a2, a3, deg_ref, b_ref, w_ref, *o_refs):
    dinv = lax.rsqrt(deg_ref[:, 0:1] + 1.0)
    a = jnp.concatenate([a0[...], a1[...], a2[...], a3[...]], axis=1)
    g = jnp.maximum(a * dinv + b_ref[...], 0.0)
    h = jnp.dot(g, w_ref[...], preferred_element_type=jnp.float32)
    _split_q(h * dinv, o_refs)


def _tmid(aq, deg, b, w):
    return pl.pallas_call(
        _tmid_body,
        grid=(_GRID,),
        in_specs=[
            *_QOUT_SPECS,
            pl.BlockSpec((_RB, 16), lambda i: (i, 0)),
            pl.BlockSpec((1, D), lambda i: (0, 0)),
            pl.BlockSpec((D, D), lambda i: (0, 0)),
        ],
        out_specs=_QOUT_SPECS,
        out_shape=_QOUT,
    )(*aq, deg, b, w)


def _tail_body(a0, a1, a2, a3, deg_ref, b3_ref,
               wm1_ref, bm1_ref, wm2_ref, bm2_ref,
               wm3_ref, bm3_ref, wm4_ref, bm4_ref,
               wf_ref, bf_ref, out_ref):
    dinv = lax.rsqrt(deg_ref[:, 0:1] + 1.0)
    a = jnp.concatenate([a0[...], a1[...], a2[...], a3[...]], axis=1)
    g = jnp.maximum(a * dinv + b3_ref[...], 0.0)
    for w_ref, b_ref in ((wm1_ref, bm1_ref), (wm2_ref, bm2_ref),
                         (wm3_ref, bm3_ref), (wm4_ref, bm4_ref)):
        g = jnp.dot(g, w_ref[...], preferred_element_type=jnp.float32)
        g = jnp.maximum(g + b_ref[...], 0.0)
    out = jnp.dot(g, wf_ref[...], preferred_element_type=jnp.float32)
    out_ref[...] = out + bf_ref[...]


def _tail(aq, deg, b3, wm1, bm1, wm2, bm2, wm3, bm3, wm4, bm4, wf, bf):
    full = lambda r, cdim: pl.BlockSpec((r, cdim), lambda i: (0, 0))
    return pl.pallas_call(
        _tail_body,
        grid=(_GRID,),
        in_specs=[
            *_QOUT_SPECS,
            pl.BlockSpec((_RB, 16), lambda i: (i, 0)),
            full(1, D),
            full(D, D), full(1, D), full(D, D), full(1, D),
            full(D, D), full(1, D), full(D, D), full(1, D),
            full(D, DOUT), full(1, DOUT),
        ],
        out_specs=pl.BlockSpec((_RB, DOUT), lambda i: (i, 0)),
        out_shape=jax.ShapeDtypeStruct((NP, DOUT), jnp.float32),
    )(*aq, deg, b3, wm1, bm1, wm2, bm2, wm3, bm3, wm4, bm4, wf, bf)


def kernel(x, edge_index, W1, b1, W2, b2, W3, b3,
           Wm1, bm1, Wm2, bm2, Wm3, bm3, Wm4, bm4, Wf, bf):
    x_pad = jnp.pad(x, ((0, NP - N), (0, 0)))
    pad = jnp.full((EP - E,), N, dtype=jnp.int32)
    srcp = jnp.concatenate([edge_index[0], pad]).reshape(16, CH, K)
    dstp = jnp.concatenate([edge_index[1], pad]).reshape(16, CH, K)

    deg = _deg_kernel(dstp)

    hq = _t0(x_pad, W1, deg)
    aq = _conv_kernel(*hq, srcp, dstp)
    hq = _tmid(aq, deg, b1.reshape(1, D), W2)
    aq = _conv_kernel(*hq, srcp, dstp)
    hq = _tmid(aq, deg, b2.reshape(1, D), W3)
    aq = _conv_kernel(*hq, srcp, dstp)
    out = _tail(aq, deg, b3.reshape(1, D),
                Wm1, bm1.reshape(1, D), Wm2, bm2.reshape(1, D),
                Wm3, bm3.reshape(1, D), Wm4, bm4.reshape(1, D),
                Wf, bf.reshape(1, DOUT))
    return out[:N]
